# Initial kernel scaffold; baseline (speedup 1.0000x reference)
import functools

import jax
import jax.numpy as jnp
from jax import lax
from jax.experimental import pallas as pl
from jax.experimental.pallas import tpu as pltpu
from jax.experimental.pallas import tpu_sc as plsc

B = 32
N = 32768
F = 64
K = 1024
L = 16
NV = N // L


def _max_body(x_ref, o_ref):
    m = jnp.max(x_ref[...], axis=1, keepdims=True)
    o_ref[...] = jnp.where(m == 0.0, 0.0, m)


def _compute_maxes(x2d):
    rows = B * N
    blk = 8192
    out = pl.pallas_call(
        _max_body,
        grid=(rows // blk,),
        in_specs=[pl.BlockSpec((blk, F), lambda i: (i, 0))],
        out_specs=pl.BlockSpec((blk, 1), lambda i: (i, 0)),
        out_shape=jax.ShapeDtypeStruct((rows, 1), jnp.float32),
    )(x2d)
    return out.reshape(B, N)


_sc_mesh = plsc.VectorSubcoreMesh(core_axis_name="c", subcore_axis_name="s")


@functools.partial(
    pl.kernel,
    out_type=jax.ShapeDtypeStruct((B, K, F), jnp.float32),
    mesh=_sc_mesh,
    scratch_types=[
        pltpu.VMEM((N,), jnp.float32),
        pltpu.VMEM((4096,), jnp.int32),
        pltpu.VMEM((256,), jnp.int32),
        pltpu.VMEM((K,), jnp.uint32),
        pltpu.VMEM((K,), jnp.int32),
        pltpu.VMEM((K,), jnp.uint32),
        pltpu.VMEM((K,), jnp.int32),
        pltpu.VMEM((8, 128), jnp.int32),
        pltpu.VMEM((K, F), jnp.float32),
        pltpu.SemaphoreType.DMA,
    ],
)
def _sc_topk(maxes_hbm, table_hbm, out_hbm,
             maxv, hist, offs, akd, aidx, bkd, bidx, fidx, rows, sem):
    b = lax.axis_index("s") * 2 + lax.axis_index("c")
    pltpu.sync_copy(maxes_hbm.at[b], maxv)

    lanes = lax.iota(jnp.int32, 16)
    ones = jnp.ones((L,), jnp.int32)
    zeros = jnp.zeros((L,), jnp.int32)

    def kd_at(i):
        v = maxv[pl.ds(i * L, L)]
        kb = lax.bitcast_convert_type(v, jnp.uint32)
        sign = (kb >> jnp.uint32(31)) != jnp.uint32(0)
        return jnp.where(sign, kb, (~kb) & jnp.uint32(0x7FFFFFFF))

    def zero_hist():
        def zb(i, c):
            hist[pl.ds(i * L, L)] = zeros
            return c
        lax.fori_loop(0, 256, zb, 0)

    prefix = jnp.uint32(0)
    cnt_before = jnp.int32(0)
    for p in range(4):
        sh = 24 - 8 * p
        zero_hist()

        def acc(i, c, _p=p, _sh=sh, _prefix=prefix):
            kd = kd_at(i)
            d = ((kd >> jnp.uint32(_sh)) & jnp.uint32(255)).astype(jnp.int32)
            if _p == 0:
                m = jnp.ones((L,), jnp.bool_)
            else:
                m = (kd >> jnp.uint32(_sh + 8)) == (_prefix >> jnp.uint32(_sh + 8))
            plsc.addupdate_scatter(hist, [lanes * 256 + d], ones, mask=m)
            return c
        lax.fori_loop(0, NV, acc, 0)

        def chunk(e, carry):
            crun, tdig, cntb, done = carry
            tot = zeros
            for l in range(16):
                tot = tot + hist[pl.ds(l * 256 + e * L, L)]
            cum = plsc.cumsum(tot)
            reached = (crun + cum) >= K
            nbelow = jnp.sum(jnp.where(reached, 0, 1).astype(jnp.int32))
            below = jnp.sum(jnp.where(reached, 0, tot))
            found = nbelow < 16
            upd = jnp.logical_and(done == 0, found)
            tdig = jnp.where(upd, e * L + nbelow, tdig)
            cntb = jnp.where(upd, crun + below, cntb)
            crun = crun + jnp.sum(tot)
            done = jnp.where(upd, jnp.int32(1), done)
            return crun, tdig, cntb, done

        _, tdig, cntb, _ = lax.fori_loop(
            0, 16, chunk,
            (cnt_before, jnp.int32(0), jnp.int32(0), jnp.int32(0)))
        prefix = prefix | (tdig.astype(jnp.uint32) << jnp.uint32(sh))
        cnt_before = cntb

    T = prefix
    count_lt = cnt_before
    need_eq = K - count_lt

    def comp(i, carry):
        off_lt, off_eq = carry
        kd = kd_at(i)
        gidx = b * N + i * L + lanes
        m_lt = kd < T
        c = plsc.cumsum(m_lt.astype(jnp.int32))
        pos = off_lt + c - 1
        plsc.store_scatter(akd, [pos], kd, mask=m_lt)
        plsc.store_scatter(aidx, [pos], gidx, mask=m_lt)
        m_eq = kd == T
        ceq = plsc.cumsum(m_eq.astype(jnp.int32))
        m_eq = jnp.logical_and(m_eq, (off_eq + ceq) <= need_eq)
        ceq2 = plsc.cumsum(m_eq.astype(jnp.int32))
        peq = count_lt + off_eq + ceq2 - 1
        plsc.store_scatter(fidx, [peq >> 7, peq & 127], gidx, mask=m_eq)
        return (off_lt + jnp.sum(m_lt.astype(jnp.int32)),
                off_eq + jnp.sum(m_eq.astype(jnp.int32)))

    lax.fori_loop(0, NV, comp, (jnp.int32(0), jnp.int32(0)))

    nv_lt = (count_lt + (L - 1)) // L
    bufs = [(akd, aidx), (bkd, bidx)]
    for p in range(4):
        sh = 8 * p
        skd, sidx = bufs[p % 2]
        dkd, didx = bufs[(p + 1) % 2]
        zero_hist()

        def hacc(i, c, _sh=sh, _skd=skd):
            m = (i * L + lanes) < count_lt
            kv = _skd[pl.ds(i * L, L)]
            d = ((kv >> jnp.uint32(_sh)) & jnp.uint32(255)).astype(jnp.int32)
            plsc.addupdate_scatter(hist, [lanes * 256 + d], ones, mask=m)
            return c
        lax.fori_loop(0, nv_lt, hacc, 0)

        def offb(e, cin):
            tot = zeros
            for l in range(16):
                tot = tot + hist[pl.ds(l * 256 + e * L, L)]
            cum = plsc.cumsum(tot)
            offs[pl.ds(e * L, L)] = cin + cum - tot
            return cin + jnp.sum(tot)
        lax.fori_loop(0, 16, offb, jnp.int32(0))

        def scat(i, c, _p=p, _sh=sh, _skd=skd, _sidx=sidx, _dkd=dkd, _didx=didx):
            m = (i * L + lanes) < count_lt
            kv = _skd[pl.ds(i * L, L)]
            iv = _sidx[pl.ds(i * L, L)]
            d = ((kv >> jnp.uint32(_sh)) & jnp.uint32(255)).astype(jnp.int32)
            base = plsc.load_gather(offs, [d])
            dup, lastm = plsc.scan_count(d, mask=m)
            pos = base + dup
            if _p == 3:
                plsc.store_scatter(fidx, [pos >> 7, pos & 127], iv, mask=m)
            else:
                plsc.store_scatter(_dkd, [pos], kv, mask=m)
                plsc.store_scatter(_didx, [pos], iv, mask=m)
            plsc.addupdate_scatter(offs, [d], dup + 1,
                                   mask=jnp.logical_and(lastm, m))
            return c
        lax.fori_loop(0, nv_lt, scat, 0)

    copies = [
        pltpu.async_copy(table_hbm.at[fidx.at[j]],
                         rows.at[pl.ds(j * 128, 128)], sem)
        for j in range(8)
    ]
    for cp in copies:
        cp.wait()
    pltpu.sync_copy(rows, out_hbm.at[b])


def kernel(output_of_dgcnn_layer):
    x2d = output_of_dgcnn_layer.reshape(B * N, F)
    maxes = _compute_maxes(x2d)
    return _sc_topk(maxes, x2d)


# speedup vs baseline: 1.0675x; 1.0675x over previous
import functools

import jax
import jax.numpy as jnp
from jax import lax
from jax.experimental import pallas as pl
from jax.experimental.pallas import tpu as pltpu
from jax.experimental.pallas import tpu_sc as plsc

B = 32
N = 32768
F = 64
K = 1024
L = 16
NV = N // L


def _max_body(x_ref, o_ref):
    m = jnp.max(x_ref[...], axis=1, keepdims=True)
    o_ref[...] = jnp.where(m == 0.0, 0.0, m)


def _compute_maxes(x2d):
    rows = B * N
    blk = 8192
    out = pl.pallas_call(
        _max_body,
        grid=(rows // blk,),
        in_specs=[pl.BlockSpec((blk, F), lambda i: (i, 0))],
        out_specs=pl.BlockSpec((blk, 1), lambda i: (i, 0)),
        out_shape=jax.ShapeDtypeStruct((rows, 1), jnp.float32),
    )(x2d)
    return out.reshape(B, N)


_sc_mesh = plsc.VectorSubcoreMesh(core_axis_name="c", subcore_axis_name="s")


@functools.partial(
    pl.kernel,
    out_type=jax.ShapeDtypeStruct((B, K, F), jnp.float32),
    mesh=_sc_mesh,
    compiler_params=pltpu.CompilerParams(needs_layout_passes=False,
                                         use_tc_tiling_on_sc=False),
    scratch_types=[
        pltpu.VMEM((N,), jnp.float32),
        pltpu.VMEM((4096,), jnp.int32),
        pltpu.VMEM((256,), jnp.int32),
        pltpu.VMEM((K,), jnp.int32),
        pltpu.VMEM((K,), jnp.int32),
        pltpu.VMEM((K,), jnp.int32),
        pltpu.VMEM((K,), jnp.int32),
        pltpu.VMEM((K,), jnp.int32),
        pltpu.VMEM((K, F), jnp.float32),
        pltpu.SemaphoreType.DMA,
    ],
)
def _sc_topk(maxes_hbm, table_hbm, out_hbm,
             maxv, hist, offs, akd, aidx, bkd, bidx, fidx, rows, sem):
    b = lax.axis_index("s") * 2 + lax.axis_index("c")
    pltpu.sync_copy(maxes_hbm.at[b], maxv)

    def zf(i, c):
        fidx[pl.ds(i * L, L)] = jnp.zeros((L,), jnp.int32)
        return c
    lax.fori_loop(0, K // L, zf, 0)

    lanes = lax.iota(jnp.int32, 16)
    ones = jnp.ones((L,), jnp.int32)
    zeros = jnp.zeros((L,), jnp.int32)

    def srl(x, s):
        return lax.shift_right_logical(
            x, jnp.full(jnp.shape(x), s, jnp.int32))

    def kd_at(i):
        v = maxv[pl.ds(i * L, L)]
        kb = lax.bitcast_convert_type(v, jnp.int32)
        return jnp.where(kb < 0, kb, (~kb) & jnp.int32(0x7FFFFFFF))

    def zero_hist():
        def zb(i, c):
            hist[pl.ds(i * L, L)] = zeros
            return c
        lax.fori_loop(0, 256, zb, 0)

    prefix = jnp.int32(0)
    cnt_before = jnp.int32(0)
    for p in range(4):
        sh = 24 - 8 * p
        zero_hist()

        def acc(i, c, _p=p, _sh=sh, _prefix=prefix):
            kd = kd_at(i)
            d = srl(kd, _sh) & jnp.int32(255)
            if _p == 0:
                m = jnp.ones((L,), jnp.bool_)
            else:
                m = srl(kd, _sh + 8) == srl(_prefix, _sh + 8)
            plsc.addupdate_scatter(hist, [lanes * 256 + d], ones, mask=m)
            return c
        lax.fori_loop(0, NV, acc, 0)

        def chunk(e, carry):
            crun, tdig, cntb, done = carry
            tot = zeros
            for l in range(16):
                tot = tot + hist[pl.ds(l * 256 + e * L, L)]
            cum = plsc.cumsum(tot)
            reached = (crun + cum) >= K
            nbelow = jnp.sum(jnp.where(reached, 0, 1).astype(jnp.int32))
            below = jnp.sum(jnp.where(reached, 0, tot))
            found = nbelow < 16
            upd = jnp.logical_and(done == 0, found)
            tdig = jnp.where(upd, e * L + nbelow, tdig)
            cntb = jnp.where(upd, crun + below, cntb)
            crun = crun + jnp.sum(tot)
            done = jnp.where(upd, jnp.int32(1), done)
            return crun, tdig, cntb, done

        _, tdig, cntb, _ = lax.fori_loop(
            0, 16, chunk,
            (cnt_before, jnp.int32(0), jnp.int32(0), jnp.int32(0)))
        prefix = prefix | (tdig << sh)
        cnt_before = cntb

    T = prefix
    count_lt = cnt_before
    need_eq = K - count_lt
    MIN32 = jnp.int32(-2147483648)
    Tx = T ^ MIN32

    def comp(i, carry):
        off_lt, off_eq = carry
        kd = kd_at(i)
        gidx = b * N + i * L + lanes
        m_lt = (kd ^ MIN32) < Tx
        c = plsc.cumsum(m_lt.astype(jnp.int32))
        pos = off_lt + c - 1
        plsc.store_scatter(akd, [pos], kd, mask=m_lt)
        plsc.store_scatter(aidx, [pos], gidx, mask=m_lt)
        m_eq = kd == T
        ceq = plsc.cumsum(m_eq.astype(jnp.int32))
        m_eq = jnp.logical_and(m_eq, (off_eq + ceq) <= need_eq)
        ceq2 = plsc.cumsum(m_eq.astype(jnp.int32))
        peq = count_lt + off_eq + ceq2 - 1
        plsc.store_scatter(fidx, [peq], gidx, mask=m_eq)
        return (off_lt + jnp.sum(m_lt.astype(jnp.int32)),
                off_eq + jnp.sum(m_eq.astype(jnp.int32)))

    lax.fori_loop(0, NV, comp, (jnp.int32(0), jnp.int32(0)))

    nv_lt = (count_lt + (L - 1)) // L
    bufs = [(akd, aidx), (bkd, bidx)]
    for p in range(4):
        sh = 8 * p
        skd, sidx = bufs[p % 2]
        dkd, didx = bufs[(p + 1) % 2]
        zero_hist()

        def hacc(i, c, _sh=sh, _skd=skd):
            m = (i * L + lanes) < count_lt
            kv = _skd[pl.ds(i * L, L)]
            d = srl(kv, _sh) & jnp.int32(255)
            plsc.addupdate_scatter(hist, [lanes * 256 + d], ones, mask=m)
            return c
        lax.fori_loop(0, nv_lt, hacc, 0)

        def offb(e, cin):
            tot = zeros
            for l in range(16):
                tot = tot + hist[pl.ds(l * 256 + e * L, L)]
            cum = plsc.cumsum(tot)
            offs[pl.ds(e * L, L)] = cin + cum - tot
            return cin + jnp.sum(tot)
        lax.fori_loop(0, 16, offb, jnp.int32(0))

        def scat(i, c, _p=p, _sh=sh, _skd=skd, _sidx=sidx, _dkd=dkd, _didx=didx):
            m = (i * L + lanes) < count_lt
            kv = _skd[pl.ds(i * L, L)]
            iv = _sidx[pl.ds(i * L, L)]
            d = srl(kv, _sh) & jnp.int32(255)
            base = plsc.load_gather(offs, [d])
            dup, lastm = plsc.scan_count(d, mask=m)
            pos = base + dup - 1
            if _p == 3:
                plsc.store_scatter(fidx, [pos], iv, mask=m)
            else:
                plsc.store_scatter(_dkd, [pos], kv, mask=m)
                plsc.store_scatter(_didx, [pos], iv, mask=m)
            plsc.addupdate_scatter(offs, [d], dup,
                                   mask=jnp.logical_and(lastm, m))
            return c
        lax.fori_loop(0, nv_lt, scat, 0)

    def grow(i, c):
        idxv = fidx[pl.ds(i * L, L)]
        idxv = jnp.clip(idxv, 0, B * N - 1)
        pltpu.async_copy(table_hbm.at[idxv],
                         rows.at[pl.ds(i * L, L)], sem).wait()
        return c
    lax.fori_loop(0, K // L, grow, 0)
    pltpu.sync_copy(rows, out_hbm.at[b])


def kernel(output_of_dgcnn_layer):
    x2d = output_of_dgcnn_layer.reshape(B * N, F)
    maxes = _compute_maxes(x2d)
    return _sc_topk(maxes, x2d)


# speedup vs baseline: 1.4726x; 1.3795x over previous
import functools

import jax
import jax.numpy as jnp
from jax import lax
from jax.experimental import pallas as pl
from jax.experimental.pallas import tpu as pltpu
from jax.experimental.pallas import tpu_sc as plsc

B = 32
N = 32768
F = 64
K = 1024
L = 16
NV = N // L


def _max_body(x_ref, o_ref):
    m = jnp.max(x_ref[...], axis=1)
    m = jnp.where(m == 0.0, 0.0, m)
    o_ref[...] = m.reshape(o_ref.shape)


def _compute_maxes(x2d):
    rows = B * N
    blk = 16384
    out = pl.pallas_call(
        _max_body,
        grid=(rows // blk,),
        in_specs=[pl.BlockSpec((blk, F), lambda i: (i, 0))],
        out_specs=pl.BlockSpec((blk // 128, 128), lambda i: (i, 0)),
        out_shape=jax.ShapeDtypeStruct((rows // 128, 128), jnp.float32),
    )(x2d)
    return out.reshape(B, N)


_sc_mesh = plsc.VectorSubcoreMesh(core_axis_name="c", subcore_axis_name="s")


@functools.partial(
    pl.kernel,
    out_type=jax.ShapeDtypeStruct((B, K, F), jnp.float32),
    mesh=_sc_mesh,
    compiler_params=pltpu.CompilerParams(needs_layout_passes=False,
                                         use_tc_tiling_on_sc=False),
    scratch_types=[
        pltpu.VMEM((N,), jnp.float32),
        pltpu.VMEM((4096,), jnp.int32),
        pltpu.VMEM((256,), jnp.int32),
        pltpu.VMEM((K,), jnp.int32),
        pltpu.VMEM((K,), jnp.int32),
        pltpu.VMEM((K,), jnp.int32),
        pltpu.VMEM((K,), jnp.int32),
        pltpu.VMEM((K,), jnp.int32),
        pltpu.VMEM((K, F), jnp.float32),
        pltpu.SemaphoreType.DMA,
    ],
)
def _sc_topk(maxes_hbm, table_hbm, out_hbm,
             maxv, hist, offs, akd, aidx, bkd, bidx, fidx, rows, sem):
    b = lax.axis_index("s") * 2 + lax.axis_index("c")
    pltpu.sync_copy(maxes_hbm.at[b], maxv)

    def zf(i, c):
        fidx[pl.ds(i * L, L)] = jnp.zeros((L,), jnp.int32)
        return c
    lax.fori_loop(0, K // L, zf, 0)

    lanes = lax.iota(jnp.int32, 16)
    ones = jnp.ones((L,), jnp.int32)
    zeros = jnp.zeros((L,), jnp.int32)

    def srl(x, s):
        return lax.shift_right_logical(
            x, jnp.full(jnp.shape(x), s, jnp.int32))

    def kd_at(i):
        v = maxv[pl.ds(i * L, L)]
        kb = lax.bitcast_convert_type(v, jnp.int32)
        return jnp.where(kb < 0, kb, (~kb) & jnp.int32(0x7FFFFFFF))

    def zero_hist():
        def zb(i, c):
            hist[pl.ds(i * L, L)] = zeros
            return c
        lax.fori_loop(0, 256, zb, 0)

    prefix = jnp.int32(0)
    cnt_before = jnp.int32(0)
    for p in range(4):
        sh = 24 - 8 * p
        zero_hist()

        def acc(i, c, _p=p, _sh=sh, _prefix=prefix):
            kd = kd_at(i)
            d = srl(kd, _sh) & jnp.int32(255)
            if _p == 0:
                m = jnp.ones((L,), jnp.bool_)
            else:
                m = srl(kd, _sh + 8) == srl(_prefix, _sh + 8)
            plsc.addupdate_scatter(hist, [lanes * 256 + d], ones, mask=m)
            return c
        lax.fori_loop(0, NV, acc, 0)

        def chunk(e, carry):
            crun, tdig, cntb, done = carry
            tot = zeros
            for l in range(16):
                tot = tot + hist[pl.ds(l * 256 + e * L, L)]
            cum = plsc.cumsum(tot)
            reached = (crun + cum) >= K
            nbelow = jnp.sum(jnp.where(reached, 0, 1).astype(jnp.int32))
            below = jnp.sum(jnp.where(reached, 0, tot))
            found = nbelow < 16
            upd = jnp.logical_and(done == 0, found)
            tdig = jnp.where(upd, e * L + nbelow, tdig)
            cntb = jnp.where(upd, crun + below, cntb)
            crun = crun + jnp.sum(tot)
            done = jnp.where(upd, jnp.int32(1), done)
            return crun, tdig, cntb, done

        _, tdig, cntb, _ = lax.fori_loop(
            0, 16, chunk,
            (cnt_before, jnp.int32(0), jnp.int32(0), jnp.int32(0)))
        prefix = prefix | (tdig << sh)
        cnt_before = cntb

    T = prefix
    count_lt = cnt_before
    need_eq = K - count_lt
    MIN32 = jnp.int32(-2147483648)
    Tx = T ^ MIN32

    def comp(i, carry):
        off_lt, off_eq = carry
        kd = kd_at(i)
        gidx = b * N + i * L + lanes
        m_lt = (kd ^ MIN32) < Tx
        c = plsc.cumsum(m_lt.astype(jnp.int32))
        pos = off_lt + c - 1
        plsc.store_scatter(akd, [pos], kd, mask=m_lt)
        plsc.store_scatter(aidx, [pos], gidx, mask=m_lt)
        m_eq = kd == T
        ceq = plsc.cumsum(m_eq.astype(jnp.int32))
        m_eq = jnp.logical_and(m_eq, (off_eq + ceq) <= need_eq)
        ceq2 = plsc.cumsum(m_eq.astype(jnp.int32))
        peq = count_lt + off_eq + ceq2 - 1
        plsc.store_scatter(fidx, [peq], gidx, mask=m_eq)
        return (off_lt + jnp.sum(m_lt.astype(jnp.int32)),
                off_eq + jnp.sum(m_eq.astype(jnp.int32)))

    lax.fori_loop(0, NV, comp, (jnp.int32(0), jnp.int32(0)))

    nv_lt = (count_lt + (L - 1)) // L
    bufs = [(akd, aidx), (bkd, bidx)]
    for p in range(4):
        sh = 8 * p
        skd, sidx = bufs[p % 2]
        dkd, didx = bufs[(p + 1) % 2]
        zero_hist()

        def hacc(i, c, _sh=sh, _skd=skd):
            m = (i * L + lanes) < count_lt
            kv = _skd[pl.ds(i * L, L)]
            d = srl(kv, _sh) & jnp.int32(255)
            plsc.addupdate_scatter(hist, [lanes * 256 + d], ones, mask=m)
            return c
        lax.fori_loop(0, nv_lt, hacc, 0)

        def offb(e, cin):
            tot = zeros
            for l in range(16):
                tot = tot + hist[pl.ds(l * 256 + e * L, L)]
            cum = plsc.cumsum(tot)
            offs[pl.ds(e * L, L)] = cin + cum - tot
            return cin + jnp.sum(tot)
        lax.fori_loop(0, 16, offb, jnp.int32(0))

        def scat(i, c, _p=p, _sh=sh, _skd=skd, _sidx=sidx, _dkd=dkd, _didx=didx):
            m = (i * L + lanes) < count_lt
            kv = _skd[pl.ds(i * L, L)]
            iv = _sidx[pl.ds(i * L, L)]
            d = srl(kv, _sh) & jnp.int32(255)
            base = plsc.load_gather(offs, [d])
            dup, lastm = plsc.scan_count(d, mask=m)
            pos = base + dup - 1
            if _p == 3:
                plsc.store_scatter(fidx, [pos], iv, mask=m)
            else:
                plsc.store_scatter(_dkd, [pos], kv, mask=m)
                plsc.store_scatter(_didx, [pos], iv, mask=m)
            plsc.addupdate_scatter(offs, [d], dup,
                                   mask=jnp.logical_and(lastm, m))
            return c
        lax.fori_loop(0, nv_lt, scat, 0)

    def grow(i, c):
        idxv = fidx[pl.ds(i * L, L)]
        idxv = jnp.clip(idxv, 0, B * N - 1)
        pltpu.async_copy(table_hbm.at[idxv],
                         rows.at[pl.ds(i * L, L)], sem).wait()
        return c
    lax.fori_loop(0, K // L, grow, 0)
    pltpu.sync_copy(rows, out_hbm.at[b])


def kernel(output_of_dgcnn_layer):
    x2d = output_of_dgcnn_layer.reshape(B * N, F)
    maxes = _compute_maxes(x2d)
    return _sc_topk(maxes, x2d)
